# baseline (speedup 1.0000x reference)
import jax
import jax.numpy as jnp
from jax.experimental import pallas as pl
from jax.experimental.pallas import tpu as pltpu

E, D, F, T = 8, 768, 2048, 2048
CT = 1024
FS = 4


def _moe_kernel(xb_ref, W1_ref, b1_ref, W2_ref, b2_ref, Wr_ref, br_ref,
                out_ref, w_ref, g_ref):
    e = pl.program_id(0)

    @pl.when(e == 0)
    def _():
        logits = jnp.dot(xb_ref[...],
                         Wr_ref[...].astype(jnp.bfloat16),
                         preferred_element_type=jnp.float32) + br_ref[...]
        m = jnp.max(logits, axis=-1, keepdims=True)
        p = jnp.exp(logits - m)
        w_ref[...] = p / jnp.sum(p, axis=-1, keepdims=True)

    pa = jax.lax.rem(e, 2)
    pb = jax.lax.rem(e + 1, 2)
    b1 = b1_ref[0]
    b2 = b2_ref[0]
    lane = jax.lax.broadcasted_iota(jnp.int32, (CT, E), 1)

    for i in range(T // CT):
        sl = pl.ds(i * CT, CT)

        @pl.when(e < E)
        def _():
            xc = xb_ref[sl, :]
            for j in range(FS):
                fs = slice(j * (F // FS), (j + 1) * (F // FS))
                h = jnp.dot(xc, W1_ref[0][:, fs].astype(jnp.bfloat16),
                            preferred_element_type=jnp.float32) + b1[:, fs]
                hb = h.astype(jnp.bfloat16)
                g_ref[pa, sl, fs] = hb * (
                    jnp.bfloat16(1.0)
                    + jax.lax.erf(hb * jnp.bfloat16(0.7071067811865476)))

        @pl.when(e > 0)
        def _():
            wc = jnp.sum(jnp.where(lane == e - 1, w_ref[sl, :], 0.0),
                         axis=1, keepdims=True)
            contrib = (0.5 * wc) * jnp.dot(
                g_ref[pb, sl, :], W2_ref[0].astype(jnp.bfloat16),
                preferred_element_type=jnp.float32) + wc * b2

            @pl.when(e == 1)
            def _():
                out_ref[sl, :] = contrib

            @pl.when(e > 1)
            def _():
                out_ref[sl, :] = out_ref[sl, :] + contrib


def kernel(x, W1, b1, W2, b2, Wr, br):
    xb = x.astype(jnp.bfloat16)
    br2 = br.reshape(1, E)
    b1r = b1.reshape(E, 1, F)
    b2r = b2.reshape(E, 1, D)
    last = E - 1
    return pl.pallas_call(
        _moe_kernel,
        grid=(E + 1,),
        in_specs=[
            pl.BlockSpec((T, D), lambda e: (0, 0)),
            pl.BlockSpec((1, D, F), lambda e: (jnp.minimum(e, last), 0, 0)),
            pl.BlockSpec((1, 1, F), lambda e: (jnp.minimum(e, last), 0, 0)),
            pl.BlockSpec((1, F, D), lambda e: (jnp.maximum(e - 1, 0), 0, 0)),
            pl.BlockSpec((1, 1, D), lambda e: (jnp.maximum(e - 1, 0), 0, 0)),
            pl.BlockSpec((D, E), lambda e: (0, 0)),
            pl.BlockSpec((1, E), lambda e: (0, 0)),
        ],
        out_specs=pl.BlockSpec((T, D), lambda e: (0, 0)),
        out_shape=jax.ShapeDtypeStruct((T, D), jnp.float32),
        scratch_shapes=[pltpu.VMEM((T, E), jnp.float32),
                        pltpu.VMEM((2, T, F), jnp.bfloat16)],
        compiler_params=pltpu.CompilerParams(
            dimension_semantics=("arbitrary",),
        ),
    )(xb, W1, b1r, W2, b2r, Wr, br2)


# speedup vs baseline: 1.0050x; 1.0050x over previous
import jax
import jax.numpy as jnp
from jax.experimental import pallas as pl
from jax.experimental.pallas import tpu as pltpu

E, D, F, T = 8, 768, 2048, 2048
CT = 1024
FS = 4


def _moe_kernel(xb_ref, W1_ref, b1_ref, W2_ref, b2_ref, Wr_ref, br_ref,
                out_ref, w_ref, g_ref):
    e = pl.program_id(0)

    @pl.when(e == 0)
    def _():
        logits = jnp.dot(xb_ref[...],
                         Wr_ref[...].astype(jnp.bfloat16),
                         preferred_element_type=jnp.float32) + br_ref[...]
        m = jnp.max(logits, axis=-1, keepdims=True)
        p = jnp.exp(logits - m)
        w_ref[...] = p / jnp.sum(p, axis=-1, keepdims=True)

    pa = jax.lax.rem(e, 2)
    pb = jax.lax.rem(e + 1, 2)
    b1 = b1_ref[0]
    lane = jax.lax.broadcasted_iota(jnp.int32, (CT, E), 1)

    for i in range(T // CT):
        sl = pl.ds(i * CT, CT)

        @pl.when(e < E)
        def _():
            xc = xb_ref[sl, :]
            for j in range(FS):
                fs = slice(j * (F // FS), (j + 1) * (F // FS))
                h = jnp.dot(xc, W1_ref[0][:, fs].astype(jnp.bfloat16),
                            preferred_element_type=jnp.float32) + b1[:, fs]
                hb = h.astype(jnp.bfloat16)
                g_ref[pa, sl, fs] = hb * (
                    jnp.bfloat16(1.0)
                    + jax.lax.erf(hb * jnp.bfloat16(0.7071067811865476)))

        @pl.when(e > 0)
        def _():
            wc = jnp.sum(jnp.where(lane == e - 1, w_ref[sl, :], 0.0),
                         axis=1, keepdims=True)
            contrib = (0.5 * wc) * jnp.dot(
                g_ref[pb, sl, :], W2_ref[0].astype(jnp.bfloat16),
                preferred_element_type=jnp.float32)

            @pl.when(e == 1)
            def _():
                out_ref[sl, :] = contrib + jnp.dot(
                    w_ref[sl, :].astype(jnp.bfloat16),
                    b2_ref[...].astype(jnp.bfloat16),
                    preferred_element_type=jnp.float32)

            @pl.when(e > 1)
            def _():
                out_ref[sl, :] = out_ref[sl, :] + contrib


def kernel(x, W1, b1, W2, b2, Wr, br):
    xb = x.astype(jnp.bfloat16)
    br2 = br.reshape(1, E)
    b1r = b1.reshape(E, 1, F)
    last = E - 1
    return pl.pallas_call(
        _moe_kernel,
        grid=(E + 1,),
        in_specs=[
            pl.BlockSpec((T, D), lambda e: (0, 0)),
            pl.BlockSpec((1, D, F), lambda e: (jnp.minimum(e, last), 0, 0)),
            pl.BlockSpec((1, 1, F), lambda e: (jnp.minimum(e, last), 0, 0)),
            pl.BlockSpec((1, F, D), lambda e: (jnp.maximum(e - 1, 0), 0, 0)),
            pl.BlockSpec((E, D), lambda e: (0, 0)),
            pl.BlockSpec((D, E), lambda e: (0, 0)),
            pl.BlockSpec((1, E), lambda e: (0, 0)),
        ],
        out_specs=pl.BlockSpec((T, D), lambda e: (0, 0)),
        out_shape=jax.ShapeDtypeStruct((T, D), jnp.float32),
        scratch_shapes=[pltpu.VMEM((T, E), jnp.float32),
                        pltpu.VMEM((2, T, F), jnp.bfloat16)],
        compiler_params=pltpu.CompilerParams(
            dimension_semantics=("arbitrary",),
        ),
    )(xb, W1, b1r, W2, b2, Wr, br2)
